# Initial kernel scaffold; baseline (speedup 1.0000x reference)
import jax
import jax.numpy as jnp
from jax.experimental import pallas as pl


def kernel(x, scalar_l, vec, vector_l, edge_feat, edge_udiff, batch, Wxp1, bxp1, Wxp2, bxp2, Wep, bep, Wsl1, bsl1, Wsl2, bsl2, Wvl, Wsg1, bsg1, Wsg2, bsg2, Wvg, Wvlp, Wslp1, bslp1, Wslp2, bslp2, Wl):
    raise NotImplementedError("write your pallas kernel here")


# speedup vs baseline: 4.5396x; 4.5396x over previous
import math

import jax
import jax.numpy as jnp
from jax.experimental import pallas as pl
from jax.experimental.pallas import tpu as pltpu

_SCALE = 1.0 / 0.6
_INV3 = 1.0 / math.sqrt(3.0)
_PREC = jax.lax.Precision.HIGHEST


def _ssilu(v):
    return jax.nn.silu(v) * _SCALE


def _dot(a, b):
    return jax.lax.dot_general(
        a, b, (((1,), (0,)), ((), ())),
        preferred_element_type=jnp.float32, precision=_PREC)


def _node_body(x_ref, vec_ref, ef_ref, eu_ref, btr_ref, btc_ref,
               slt_ref, vlt_ref,
               wxp1_ref, bxp1_ref, wxp2_ref, bxp2_ref, wep_ref, bep_ref,
               wsl1a_ref, wsl1b_ref, bsl1_ref, wsl2_ref, bsl2_ref, wvl_ref,
               hx_ref, hvec_ref, psx_ref, psv_ref, pcnt_ref):
    f32 = jnp.float32
    blk, F = x_ref.shape
    B = slt_ref.shape[0]
    invh = 1.0 / math.sqrt(float(F))

    xb = x_ref[...]
    ids_c = btc_ref[0]
    ids_r = btr_ref[0]
    onehot = (ids_c == jax.lax.broadcasted_iota(jnp.int32, (1, B), 1)
              ).astype(f32)
    onehot_t = (jax.lax.broadcasted_iota(jnp.int32, (B, 1), 0) == ids_r
                ).astype(f32)

    t = _ssilu(_dot(xb, wxp1_ref[...]) + bxp1_ref[...])
    x_p = _dot(t, wxp2_ref[...]) + bxp2_ref[...]
    edge_p = _dot(ef_ref[...], wep_ref[...]) + bep_ref[...]
    prod = x_p * edge_p * _INV3
    x1 = prod[:, :F]
    x2 = prod[:, F:2 * F]
    xn = prod[:, 2 * F:] + xb

    slb = _dot(slt_ref[...], wsl1b_ref[...])
    h = _ssilu(_dot(xn, wsl1a_ref[...]) + _dot(onehot, slb) + bsl1_ref[...])
    hx = _ssilu(_dot(h, wsl2_ref[...]) + bsl2_ref[...]) + xn
    hx_ref[...] = hx
    psx_ref[0] = _dot(onehot_t, hx)
    pcnt_ref[0] = _dot(onehot_t, jnp.ones((blk, 8), f32))

    wvl = wvl_ref[...]
    for k in range(3):
        vk = vec_ref[:, k, :]
        uk = eu_ref[:, k:k + 1]
        vecn_k = (x1 * vk + x2 * uk) * invh
        vlw = _dot(vlt_ref[:, k, :], wvl)
        hvk = _dot(vecn_k, wvl) + _dot(onehot, vlw) + vecn_k
        hvec_ref[:, k, :] = hvk
        psv_ref[0, :, k * F:(k + 1) * F] = _dot(onehot_t, hvk)


def _graph_body(psx_ref, psv_ref, pcnt_ref, slt_ref, vlt_ref,
                wsg1a_ref, wsg1b_ref, bsg1_ref, wsg2_ref, bsg2_ref, wvg_ref,
                wvlp_ref, wslp1a_ref, wslp1b_ref, bslp1_ref,
                wslp2_ref, bslp2_ref, wl_ref,
                sl_ref, vl_ref, ld_ref):
    B, F = slt_ref.shape[0], slt_ref.shape[1]
    sum_x = jnp.sum(psx_ref[...], axis=0)
    sum_v = jnp.sum(psv_ref[...], axis=0)
    cnt = jnp.sum(pcnt_ref[...], axis=0)
    rinv = 1.0 / jnp.maximum(cnt[:, 0:1], 1.0)

    scalar_l = slt_ref[...]
    mean_x = sum_x * rinv
    h = _ssilu(_dot(mean_x, wsg1a_ref[...]) + _dot(scalar_l, wsg1b_ref[...])
               + bsg1_ref[...])
    sl = scalar_l + _ssilu(_dot(h, wsg2_ref[...]) + bsg2_ref[...])

    wvg = wvg_ref[...]
    wvlp = wvlp_ref[...]
    vh1 = []
    vls = []
    vn2 = jnp.zeros((B, F), jnp.float32)
    for k in range(3):
        vl_k = vlt_ref[:, k, :]
        mean_vk = sum_v[:, k * F:(k + 1) * F] * rinv
        vlk = vl_k + _dot(mean_vk + vl_k, wvg)
        vls.append(vlk)
        vh = _dot(vlk, wvlp)
        vh1.append(vh[:, :F])
        vn2 = vn2 + vh[:, F:] * vh[:, F:]
    vnorm = jnp.sqrt(vn2 + 1e-8)

    sh = _dot(_ssilu(_dot(sl, wslp1a_ref[...]) + _dot(vnorm, wslp1b_ref[...])
                     + bslp1_ref[...]), wslp2_ref[...]) + bslp2_ref[...]
    sh1 = sh[:, :F]
    sh2 = sh[:, F:2 * F]
    gate = jnp.tanh(sh[:, 2 * F:])
    sl_ref[...] = sh2 + sl * gate
    wl = wl_ref[...]
    for k in range(3):
        vlo = sh1 * vh1[k] + vls[k]
        vl_ref[:, k, :] = vlo
        ld_ref[:, k, :] = _dot(vlo, wl)


def kernel(x, scalar_l, vec, vector_l, edge_feat, edge_udiff, batch,
           Wxp1, bxp1, Wxp2, bxp2, Wep, bep, Wsl1, bsl1, Wsl2, bsl2, Wvl,
           Wsg1, bsg1, Wsg2, bsg2, Wvg, Wvlp, Wslp1, bslp1, Wslp2, bslp2, Wl):
    N, F = x.shape
    B = scalar_l.shape[0]
    R = edge_feat.shape[1]
    blk = 2000
    nb = N // blk

    bt_row = batch.reshape(nb, 1, blk)
    bt_col = batch.reshape(nb, blk, 1)
    r2 = lambda b: b.reshape(1, -1)

    rep = lambda shape: pl.BlockSpec(shape, lambda i: (0,) * len(shape))

    f32 = jnp.float32
    out_shape = [
        jax.ShapeDtypeStruct((N, F), f32),
        jax.ShapeDtypeStruct((N, 3, F), f32),
        jax.ShapeDtypeStruct((nb, B, F), f32),
        jax.ShapeDtypeStruct((nb, B, 3 * F), f32),
        jax.ShapeDtypeStruct((nb, B, 8), f32),
    ]
    in_specs = [
        pl.BlockSpec((blk, F), lambda i: (i, 0)),
        pl.BlockSpec((blk, 3, F), lambda i: (i, 0, 0)),
        pl.BlockSpec((blk, R), lambda i: (i, 0)),
        pl.BlockSpec((blk, 3), lambda i: (i, 0)),
        pl.BlockSpec((1, 1, blk), lambda i: (i, 0, 0)),
        pl.BlockSpec((1, blk, 1), lambda i: (i, 0, 0)),
        rep((B, F)), rep((B, 3, F)),
        rep((F, F)), rep((1, F)), rep((F, 3 * F)), rep((1, 3 * F)),
        rep((R, 3 * F)), rep((1, 3 * F)),
        rep((F, F)), rep((F, F)), rep((1, F)),
        rep((F, F)), rep((1, F)), rep((F, F)),
    ]
    out_specs = [
        pl.BlockSpec((blk, F), lambda i: (i, 0)),
        pl.BlockSpec((blk, 3, F), lambda i: (i, 0, 0)),
        pl.BlockSpec((1, B, F), lambda i: (i, 0, 0)),
        pl.BlockSpec((1, B, 3 * F), lambda i: (i, 0, 0)),
        pl.BlockSpec((1, B, 8), lambda i: (i, 0, 0)),
    ]
    hx, hvec, psx, psv, pcnt = pl.pallas_call(
        _node_body,
        grid=(nb,),
        in_specs=in_specs,
        out_specs=out_specs,
        out_shape=out_shape,
        compiler_params=pltpu.CompilerParams(
            dimension_semantics=("parallel",)),
    )(x, vec, edge_feat, edge_udiff, bt_row, bt_col,
      scalar_l, vector_l,
      Wxp1, r2(bxp1), Wxp2, r2(bxp2), Wep, r2(bep),
      Wsl1[:F], Wsl1[F:], r2(bsl1), Wsl2, r2(bsl2), Wvl)

    sl, vl, l_delta = pl.pallas_call(
        _graph_body,
        out_shape=[
            jax.ShapeDtypeStruct((B, F), f32),
            jax.ShapeDtypeStruct((B, 3, F), f32),
            jax.ShapeDtypeStruct((B, 3, 1), f32),
        ],
    )(psx, psv, pcnt, scalar_l, vector_l,
      Wsg1[:F], Wsg1[F:], r2(bsg1), Wsg2, r2(bsg2), Wvg,
      Wvlp, Wslp1[:F], Wslp1[F:], r2(bslp1), Wslp2, r2(bslp2), Wl)

    return (hx, hvec, sl, vl, l_delta)


# speedup vs baseline: 9.3509x; 2.0598x over previous
import math

import jax
import jax.numpy as jnp
from jax.experimental import pallas as pl
from jax.experimental.pallas import tpu as pltpu

_SCALE = 1.0 / 0.6
_INV3 = 1.0 / math.sqrt(3.0)
_PREC = jax.lax.Precision.DEFAULT


def _ssilu(v):
    return jax.nn.silu(v) * _SCALE


def _dot(a, b):
    return jax.lax.dot_general(
        a, b, (((1,), (0,)), ((), ())),
        preferred_element_type=jnp.float32, precision=_PREC)


def _node_body(x_ref, vec_ref, ef_ref, eu_ref, btr_ref, btc_ref,
               slt_ref, vlt_ref,
               wxp1_ref, bxp1_ref, wxp2_ref, bxp2_ref, wep_ref, bep_ref,
               wsl1a_ref, wsl1b_ref, bsl1_ref, wsl2_ref, bsl2_ref, wvl_ref,
               hx_ref, hvec_ref, psx_ref, psv_ref, pcnt_ref):
    f32 = jnp.float32
    blk, F = x_ref.shape
    B = slt_ref.shape[0]
    invh = 1.0 / math.sqrt(float(F))

    xb = x_ref[...]
    ids_c = btc_ref[0]
    ids_r = btr_ref[0]
    onehot = (ids_c == jax.lax.broadcasted_iota(jnp.int32, (1, B), 1)
              ).astype(f32)
    onehot_t = (jax.lax.broadcasted_iota(jnp.int32, (B, 1), 0) == ids_r
                ).astype(f32)

    t = _ssilu(_dot(xb, wxp1_ref[...]) + bxp1_ref[...])
    x_p = _dot(t, wxp2_ref[...]) + bxp2_ref[...]
    edge_p = _dot(ef_ref[...], wep_ref[...]) + bep_ref[...]
    prod = x_p * edge_p * _INV3
    x1 = prod[:, :F]
    x2 = prod[:, F:2 * F]
    xn = prod[:, 2 * F:] + xb

    slb = _dot(slt_ref[...], wsl1b_ref[...])
    h = _ssilu(_dot(xn, wsl1a_ref[...]) + _dot(onehot, slb) + bsl1_ref[...])
    hx = _ssilu(_dot(h, wsl2_ref[...]) + bsl2_ref[...]) + xn
    hx_ref[...] = hx
    psx_ref[0] = _dot(onehot_t, hx)
    pcnt_ref[0] = _dot(onehot_t, jnp.ones((blk, 8), f32))

    wvl = wvl_ref[...]
    for k in range(3):
        vk = vec_ref[:, k, :]
        uk = eu_ref[:, k:k + 1]
        vecn_k = (x1 * vk + x2 * uk) * invh
        vlw = _dot(vlt_ref[:, k, :], wvl)
        hvk = _dot(vecn_k, wvl) + _dot(onehot, vlw) + vecn_k
        hvec_ref[:, k, :] = hvk
        psv_ref[0, :, k * F:(k + 1) * F] = _dot(onehot_t, hvk)


def _graph_body(psx_ref, psv_ref, pcnt_ref, slt_ref, vlt_ref,
                wsg1a_ref, wsg1b_ref, bsg1_ref, wsg2_ref, bsg2_ref, wvg_ref,
                wvlp_ref, wslp1a_ref, wslp1b_ref, bslp1_ref,
                wslp2_ref, bslp2_ref, wl_ref,
                sl_ref, vl_ref, ld_ref):
    B, F = slt_ref.shape[0], slt_ref.shape[1]
    sum_x = jnp.sum(psx_ref[...], axis=0)
    sum_v = jnp.sum(psv_ref[...], axis=0)
    cnt = jnp.sum(pcnt_ref[...], axis=0)
    rinv = 1.0 / jnp.maximum(cnt[:, 0:1], 1.0)

    scalar_l = slt_ref[...]
    mean_x = sum_x * rinv
    h = _ssilu(_dot(mean_x, wsg1a_ref[...]) + _dot(scalar_l, wsg1b_ref[...])
               + bsg1_ref[...])
    sl = scalar_l + _ssilu(_dot(h, wsg2_ref[...]) + bsg2_ref[...])

    wvg = wvg_ref[...]
    wvlp = wvlp_ref[...]
    vh1 = []
    vls = []
    vn2 = jnp.zeros((B, F), jnp.float32)
    for k in range(3):
        vl_k = vlt_ref[:, k, :]
        mean_vk = sum_v[:, k * F:(k + 1) * F] * rinv
        vlk = vl_k + _dot(mean_vk + vl_k, wvg)
        vls.append(vlk)
        vh = _dot(vlk, wvlp)
        vh1.append(vh[:, :F])
        vn2 = vn2 + vh[:, F:] * vh[:, F:]
    vnorm = jnp.sqrt(vn2 + 1e-8)

    sh = _dot(_ssilu(_dot(sl, wslp1a_ref[...]) + _dot(vnorm, wslp1b_ref[...])
                     + bslp1_ref[...]), wslp2_ref[...]) + bslp2_ref[...]
    sh1 = sh[:, :F]
    sh2 = sh[:, F:2 * F]
    gate = jnp.tanh(sh[:, 2 * F:])
    sl_ref[...] = sh2 + sl * gate
    wl = wl_ref[...]
    for k in range(3):
        vlo = sh1 * vh1[k] + vls[k]
        vl_ref[:, k, :] = vlo
        ld_ref[:, k, :] = _dot(vlo, wl)


def kernel(x, scalar_l, vec, vector_l, edge_feat, edge_udiff, batch,
           Wxp1, bxp1, Wxp2, bxp2, Wep, bep, Wsl1, bsl1, Wsl2, bsl2, Wvl,
           Wsg1, bsg1, Wsg2, bsg2, Wvg, Wvlp, Wslp1, bslp1, Wslp2, bslp2, Wl):
    N, F = x.shape
    B = scalar_l.shape[0]
    R = edge_feat.shape[1]
    blk = 2000
    nb = N // blk

    bt_row = batch.reshape(nb, 1, blk)
    bt_col = batch.reshape(nb, blk, 1)
    r2 = lambda b: b.reshape(1, -1)

    rep = lambda shape: pl.BlockSpec(shape, lambda i: (0,) * len(shape))

    f32 = jnp.float32
    out_shape = [
        jax.ShapeDtypeStruct((N, F), f32),
        jax.ShapeDtypeStruct((N, 3, F), f32),
        jax.ShapeDtypeStruct((nb, B, F), f32),
        jax.ShapeDtypeStruct((nb, B, 3 * F), f32),
        jax.ShapeDtypeStruct((nb, B, 8), f32),
    ]
    in_specs = [
        pl.BlockSpec((blk, F), lambda i: (i, 0)),
        pl.BlockSpec((blk, 3, F), lambda i: (i, 0, 0)),
        pl.BlockSpec((blk, R), lambda i: (i, 0)),
        pl.BlockSpec((blk, 3), lambda i: (i, 0)),
        pl.BlockSpec((1, 1, blk), lambda i: (i, 0, 0)),
        pl.BlockSpec((1, blk, 1), lambda i: (i, 0, 0)),
        rep((B, F)), rep((B, 3, F)),
        rep((F, F)), rep((1, F)), rep((F, 3 * F)), rep((1, 3 * F)),
        rep((R, 3 * F)), rep((1, 3 * F)),
        rep((F, F)), rep((F, F)), rep((1, F)),
        rep((F, F)), rep((1, F)), rep((F, F)),
    ]
    out_specs = [
        pl.BlockSpec((blk, F), lambda i: (i, 0)),
        pl.BlockSpec((blk, 3, F), lambda i: (i, 0, 0)),
        pl.BlockSpec((1, B, F), lambda i: (i, 0, 0)),
        pl.BlockSpec((1, B, 3 * F), lambda i: (i, 0, 0)),
        pl.BlockSpec((1, B, 8), lambda i: (i, 0, 0)),
    ]
    hx, hvec, psx, psv, pcnt = pl.pallas_call(
        _node_body,
        grid=(nb,),
        in_specs=in_specs,
        out_specs=out_specs,
        out_shape=out_shape,
        compiler_params=pltpu.CompilerParams(
            dimension_semantics=("parallel",)),
    )(x, vec, edge_feat, edge_udiff, bt_row, bt_col,
      scalar_l, vector_l,
      Wxp1, r2(bxp1), Wxp2, r2(bxp2), Wep, r2(bep),
      Wsl1[:F], Wsl1[F:], r2(bsl1), Wsl2, r2(bsl2), Wvl)

    sl, vl, l_delta = pl.pallas_call(
        _graph_body,
        out_shape=[
            jax.ShapeDtypeStruct((B, F), f32),
            jax.ShapeDtypeStruct((B, 3, F), f32),
            jax.ShapeDtypeStruct((B, 3, 1), f32),
        ],
    )(psx, psv, pcnt, scalar_l, vector_l,
      Wsg1[:F], Wsg1[F:], r2(bsg1), Wsg2, r2(bsg2), Wvg,
      Wvlp, Wslp1[:F], Wslp1[F:], r2(bslp1), Wslp2, r2(bslp2), Wl)

    return (hx, hvec, sl, vl, l_delta)


# speedup vs baseline: 10.4805x; 1.1208x over previous
import math

import jax
import jax.numpy as jnp
from jax.experimental import pallas as pl
from jax.experimental.pallas import tpu as pltpu

_SCALE = 1.0 / 0.6
_INV3 = 1.0 / math.sqrt(3.0)
_PREC = jax.lax.Precision.DEFAULT


def _ssilu(v):
    return jax.nn.silu(v) * _SCALE


def _dot(a, b):
    return jax.lax.dot_general(
        a, b, (((1,), (0,)), ((), ())),
        preferred_element_type=jnp.float32, precision=_PREC)


def _node_body(x_ref, vec_ref, ef_ref, eu_ref, btr_ref, btc_ref,
               slt_ref, vlt_ref,
               wxp1_ref, bxp1_ref, wxp2_ref, bxp2_ref, wep_ref, bep_ref,
               wsl1a_ref, wsl1b_ref, bsl1_ref, wsl2_ref, bsl2_ref, wvl_ref,
               hx_ref, hvec_ref, psx_ref, psv_ref, pcnt_ref):
    f32 = jnp.float32
    blk, F = x_ref.shape
    B = slt_ref.shape[0]
    invh = 1.0 / math.sqrt(float(F))

    xb = x_ref[...]
    ids_c = btc_ref[0]
    ids_r = btr_ref[0]
    onehot = (ids_c == jax.lax.broadcasted_iota(jnp.int32, (1, B), 1)
              ).astype(f32)
    onehot_t = (jax.lax.broadcasted_iota(jnp.int32, (B, 1), 0) == ids_r
                ).astype(f32)

    t = _ssilu(_dot(xb, wxp1_ref[...]) + bxp1_ref[...])
    x_p = _dot(t, wxp2_ref[...]) + bxp2_ref[...]
    edge_p = _dot(ef_ref[...], wep_ref[...]) + bep_ref[...]
    prod = x_p * edge_p * _INV3
    x1 = prod[:, :F]
    x2 = prod[:, F:2 * F]
    xn = prod[:, 2 * F:] + xb

    slb = _dot(slt_ref[...], wsl1b_ref[...])
    h = _ssilu(_dot(xn, wsl1a_ref[...]) + _dot(onehot, slb) + bsl1_ref[...])
    hx = _ssilu(_dot(h, wsl2_ref[...]) + bsl2_ref[...]) + xn
    hx_ref[...] = hx
    psx_ref[0] = _dot(onehot_t, hx)
    pcnt_ref[0] = _dot(onehot_t, jnp.ones((blk, 8), f32))

    wvl = wvl_ref[...]
    for k in range(3):
        vk = vec_ref[:, k * F:(k + 1) * F]
        uk = eu_ref[:, k:k + 1]
        vecn_k = (x1 * vk + x2 * uk) * invh
        vlw = _dot(vlt_ref[:, k * F:(k + 1) * F], wvl)
        hvk = _dot(vecn_k, wvl) + _dot(onehot, vlw) + vecn_k
        hvec_ref[:, k * F:(k + 1) * F] = hvk
        psv_ref[0, :, k * F:(k + 1) * F] = _dot(onehot_t, hvk)


def _graph_body(psx_ref, psv_ref, pcnt_ref, slt_ref, vlt_ref,
                wsg1a_ref, wsg1b_ref, bsg1_ref, wsg2_ref, bsg2_ref, wvg_ref,
                wvlp_ref, wslp1a_ref, wslp1b_ref, bslp1_ref,
                wslp2_ref, bslp2_ref, wl_ref,
                sl_ref, vl_ref, ld_ref):
    B, F = slt_ref.shape[0], slt_ref.shape[1]
    sum_x = jnp.sum(psx_ref[...], axis=0)
    sum_v = jnp.sum(psv_ref[...], axis=0)
    cnt = jnp.sum(pcnt_ref[...], axis=0)
    rinv = 1.0 / jnp.maximum(cnt[:, 0:1], 1.0)

    scalar_l = slt_ref[...]
    mean_x = sum_x * rinv
    h = _ssilu(_dot(mean_x, wsg1a_ref[...]) + _dot(scalar_l, wsg1b_ref[...])
               + bsg1_ref[...])
    sl = scalar_l + _ssilu(_dot(h, wsg2_ref[...]) + bsg2_ref[...])

    wvg = wvg_ref[...]
    wvlp = wvlp_ref[...]
    vh1 = []
    vls = []
    vn2 = jnp.zeros((B, F), jnp.float32)
    for k in range(3):
        vl_k = vlt_ref[:, k, :]
        mean_vk = sum_v[:, k * F:(k + 1) * F] * rinv
        vlk = vl_k + _dot(mean_vk + vl_k, wvg)
        vls.append(vlk)
        vh = _dot(vlk, wvlp)
        vh1.append(vh[:, :F])
        vn2 = vn2 + vh[:, F:] * vh[:, F:]
    vnorm = jnp.sqrt(vn2 + 1e-8)

    sh = _dot(_ssilu(_dot(sl, wslp1a_ref[...]) + _dot(vnorm, wslp1b_ref[...])
                     + bslp1_ref[...]), wslp2_ref[...]) + bslp2_ref[...]
    sh1 = sh[:, :F]
    sh2 = sh[:, F:2 * F]
    gate = jnp.tanh(sh[:, 2 * F:])
    sl_ref[...] = sh2 + sl * gate
    wl = wl_ref[...]
    for k in range(3):
        vlo = sh1 * vh1[k] + vls[k]
        vl_ref[:, k, :] = vlo
        ld_ref[:, k, :] = _dot(vlo, wl)


def kernel(x, scalar_l, vec, vector_l, edge_feat, edge_udiff, batch,
           Wxp1, bxp1, Wxp2, bxp2, Wep, bep, Wsl1, bsl1, Wsl2, bsl2, Wvl,
           Wsg1, bsg1, Wsg2, bsg2, Wvg, Wvlp, Wslp1, bslp1, Wslp2, bslp2, Wl):
    N, F = x.shape
    B = scalar_l.shape[0]
    R = edge_feat.shape[1]
    blk = 2000
    nb = N // blk

    bt_row = batch.reshape(nb, 1, blk)
    bt_col = batch.reshape(nb, blk, 1)
    r2 = lambda b: b.reshape(1, -1)

    rep = lambda shape: pl.BlockSpec(shape, lambda i: (0,) * len(shape))

    f32 = jnp.float32
    out_shape = [
        jax.ShapeDtypeStruct((N, F), f32),
        jax.ShapeDtypeStruct((N, 3 * F), f32),
        jax.ShapeDtypeStruct((nb, B, F), f32),
        jax.ShapeDtypeStruct((nb, B, 3 * F), f32),
        jax.ShapeDtypeStruct((nb, B, 8), f32),
    ]
    in_specs = [
        pl.BlockSpec((blk, F), lambda i: (i, 0)),
        pl.BlockSpec((blk, 3 * F), lambda i: (i, 0)),
        pl.BlockSpec((blk, R), lambda i: (i, 0)),
        pl.BlockSpec((blk, 3), lambda i: (i, 0)),
        pl.BlockSpec((1, 1, blk), lambda i: (i, 0, 0)),
        pl.BlockSpec((1, blk, 1), lambda i: (i, 0, 0)),
        rep((B, F)), rep((B, 3 * F)),
        rep((F, F)), rep((1, F)), rep((F, 3 * F)), rep((1, 3 * F)),
        rep((R, 3 * F)), rep((1, 3 * F)),
        rep((F, F)), rep((F, F)), rep((1, F)),
        rep((F, F)), rep((1, F)), rep((F, F)),
    ]
    out_specs = [
        pl.BlockSpec((blk, F), lambda i: (i, 0)),
        pl.BlockSpec((blk, 3 * F), lambda i: (i, 0)),
        pl.BlockSpec((1, B, F), lambda i: (i, 0, 0)),
        pl.BlockSpec((1, B, 3 * F), lambda i: (i, 0, 0)),
        pl.BlockSpec((1, B, 8), lambda i: (i, 0, 0)),
    ]
    hx, hvf, psx, psv, pcnt = pl.pallas_call(
        _node_body,
        grid=(nb,),
        in_specs=in_specs,
        out_specs=out_specs,
        out_shape=out_shape,
        compiler_params=pltpu.CompilerParams(
            dimension_semantics=("parallel",)),
    )(x, vec.reshape(N, 3 * F), edge_feat, edge_udiff, bt_row, bt_col,
      scalar_l, vector_l.reshape(B, 3 * F),
      Wxp1, r2(bxp1), Wxp2, r2(bxp2), Wep, r2(bep),
      Wsl1[:F], Wsl1[F:], r2(bsl1), Wsl2, r2(bsl2), Wvl)
    hvec = hvf.reshape(N, 3, F)

    sl, vl, l_delta = pl.pallas_call(
        _graph_body,
        out_shape=[
            jax.ShapeDtypeStruct((B, F), f32),
            jax.ShapeDtypeStruct((B, 3, F), f32),
            jax.ShapeDtypeStruct((B, 3, 1), f32),
        ],
    )(psx, psv, pcnt, scalar_l, vector_l,
      Wsg1[:F], Wsg1[F:], r2(bsg1), Wsg2, r2(bsg2), Wvg,
      Wvlp, Wslp1[:F], Wslp1[F:], r2(bslp1), Wslp2, r2(bslp2), Wl)

    return (hx, hvec, sl, vl, l_delta)


# speedup vs baseline: 27.2936x; 2.6042x over previous
import math

import jax
import jax.numpy as jnp
from jax.experimental import pallas as pl
from jax.experimental.pallas import tpu as pltpu

_SCALE = 1.0 / 0.6
_INV3 = 1.0 / math.sqrt(3.0)
_PREC = jax.lax.Precision.DEFAULT


def _ssilu(v):
    return jax.nn.silu(v) * _SCALE


def _dot(a, b):
    return jax.lax.dot_general(
        a, b, (((1,), (0,)), ((), ())),
        preferred_element_type=jnp.float32, precision=_PREC)


def _dot_tl(a, b):
    return jax.lax.dot_general(
        a, b, (((0,), (0,)), ((), ())),
        preferred_element_type=jnp.float32, precision=_PREC)


def _node_body(x_ref, vec_ref, ef_ref, eu_ref, btr_ref,
               slt_ref, vlt_ref,
               wxp1_ref, bxp1_ref, wxp2_ref, bxp2_ref, wep_ref, bep_ref,
               wsl1a_ref, wsl1b_ref, bsl1_ref, wsl2_ref, bsl2_ref, wvl_ref,
               hx_ref, hvt_ref, psx_ref, psv_ref, pcnt_ref):
    f32 = jnp.float32
    blk, F = x_ref.shape
    B = slt_ref.shape[0]
    invh = 1.0 / math.sqrt(float(F))

    xb = x_ref[...]
    ids_r = btr_ref[0]
    onehot_t = (jax.lax.broadcasted_iota(jnp.int32, (B, 1), 0) == ids_r
                ).astype(f32)

    t = _ssilu(_dot(xb, wxp1_ref[...]) + bxp1_ref[...])
    x_p = _dot(t, wxp2_ref[...]) + bxp2_ref[...]
    edge_p = _dot(ef_ref[...], wep_ref[...]) + bep_ref[...]
    prod = x_p * edge_p * _INV3
    x1 = prod[:, :F]
    x2 = prod[:, F:2 * F]
    xn = prod[:, 2 * F:] + xb

    slb = _dot(slt_ref[...], wsl1b_ref[...])
    h = _ssilu(_dot(xn, wsl1a_ref[...]) + _dot_tl(onehot_t, slb)
               + bsl1_ref[...])
    hx = _ssilu(_dot(h, wsl2_ref[...]) + bsl2_ref[...]) + xn
    hx_ref[...] = hx
    psx_ref[0] = _dot(onehot_t, hx)
    pcnt_ref[0] = _dot(onehot_t, jnp.ones((blk, 8), f32))

    wvl = wvl_ref[...]
    for k in range(3):
        vk = vec_ref[k]
        uk = eu_ref[:, k:k + 1]
        vecn_k = (x1 * vk + x2 * uk) * invh
        vlw = _dot(vlt_ref[k], wvl)
        hvk = _dot(vecn_k, wvl) + _dot_tl(onehot_t, vlw) + vecn_k
        hvt_ref[k] = hvk
        psv_ref[0, :, k * F:(k + 1) * F] = _dot(onehot_t, hvk)


def _graph_body(psx_ref, psv_ref, pcnt_ref, slt_ref, vlt_ref,
                wsg1a_ref, wsg1b_ref, bsg1_ref, wsg2_ref, bsg2_ref, wvg_ref,
                wvlp_ref, wslp1a_ref, wslp1b_ref, bslp1_ref,
                wslp2_ref, bslp2_ref, wl_ref,
                sl_ref, vlo_ref, ld_ref):
    B, F = slt_ref.shape[0], slt_ref.shape[1]
    sum_x = jnp.sum(psx_ref[...], axis=0)
    sum_v = jnp.sum(psv_ref[...], axis=0)
    cnt = jnp.sum(pcnt_ref[...], axis=0)
    rinv = 1.0 / jnp.maximum(cnt[:, 0:1], 1.0)

    scalar_l = slt_ref[...]
    mean_x = sum_x * rinv
    h = _ssilu(_dot(mean_x, wsg1a_ref[...]) + _dot(scalar_l, wsg1b_ref[...])
               + bsg1_ref[...])
    sl = scalar_l + _ssilu(_dot(h, wsg2_ref[...]) + bsg2_ref[...])

    wvg = wvg_ref[...]
    wvlp = wvlp_ref[...]
    vh1 = []
    vls = []
    vn2 = jnp.zeros((B, F), jnp.float32)
    for k in range(3):
        vl_k = vlt_ref[k]
        mean_vk = sum_v[:, k * F:(k + 1) * F] * rinv
        vlk = vl_k + _dot(mean_vk + vl_k, wvg)
        vls.append(vlk)
        vh = _dot(vlk, wvlp)
        vh1.append(vh[:, :F])
        vn2 = vn2 + vh[:, F:] * vh[:, F:]
    vnorm = jnp.sqrt(vn2 + 1e-8)

    sh = _dot(_ssilu(_dot(sl, wslp1a_ref[...]) + _dot(vnorm, wslp1b_ref[...])
                     + bslp1_ref[...]), wslp2_ref[...]) + bslp2_ref[...]
    sh1 = sh[:, :F]
    sh2 = sh[:, F:2 * F]
    gate = jnp.tanh(sh[:, 2 * F:])
    sl_ref[...] = sh2 + sl * gate
    wl = wl_ref[...]
    for k in range(3):
        vlo = sh1 * vh1[k] + vls[k]
        vlo_ref[k] = vlo
        ld_ref[k] = _dot(vlo, wl)


def kernel(x, scalar_l, vec, vector_l, edge_feat, edge_udiff, batch,
           Wxp1, bxp1, Wxp2, bxp2, Wep, bep, Wsl1, bsl1, Wsl2, bsl2, Wvl,
           Wsg1, bsg1, Wsg2, bsg2, Wvg, Wvlp, Wslp1, bslp1, Wslp2, bslp2, Wl):
    N, F = x.shape
    B = scalar_l.shape[0]
    R = edge_feat.shape[1]
    blk = 2000
    nb = N // blk

    vec_t = jnp.transpose(vec, (1, 0, 2))
    vlt_t = jnp.transpose(vector_l, (1, 0, 2))
    bt_row = batch.reshape(nb, 1, blk)
    r2 = lambda b: b.reshape(1, -1)

    rep = lambda shape: pl.BlockSpec(shape, lambda i: (0,) * len(shape))

    f32 = jnp.float32
    out_shape = [
        jax.ShapeDtypeStruct((N, F), f32),
        jax.ShapeDtypeStruct((3, N, F), f32),
        jax.ShapeDtypeStruct((nb, B, F), f32),
        jax.ShapeDtypeStruct((nb, B, 3 * F), f32),
        jax.ShapeDtypeStruct((nb, B, 8), f32),
    ]
    in_specs = [
        pl.BlockSpec((blk, F), lambda i: (i, 0)),
        pl.BlockSpec((3, blk, F), lambda i: (0, i, 0)),
        pl.BlockSpec((blk, R), lambda i: (i, 0)),
        pl.BlockSpec((blk, 3), lambda i: (i, 0)),
        pl.BlockSpec((1, 1, blk), lambda i: (i, 0, 0)),
        rep((B, F)), rep((3, B, F)),
        rep((F, F)), rep((1, F)), rep((F, 3 * F)), rep((1, 3 * F)),
        rep((R, 3 * F)), rep((1, 3 * F)),
        rep((F, F)), rep((F, F)), rep((1, F)),
        rep((F, F)), rep((1, F)), rep((F, F)),
    ]
    out_specs = [
        pl.BlockSpec((blk, F), lambda i: (i, 0)),
        pl.BlockSpec((3, blk, F), lambda i: (0, i, 0)),
        pl.BlockSpec((1, B, F), lambda i: (i, 0, 0)),
        pl.BlockSpec((1, B, 3 * F), lambda i: (i, 0, 0)),
        pl.BlockSpec((1, B, 8), lambda i: (i, 0, 0)),
    ]
    hx, hvt, psx, psv, pcnt = pl.pallas_call(
        _node_body,
        grid=(nb,),
        in_specs=in_specs,
        out_specs=out_specs,
        out_shape=out_shape,
        compiler_params=pltpu.CompilerParams(
            dimension_semantics=("parallel",)),
    )(x, vec_t, edge_feat, edge_udiff, bt_row,
      scalar_l, vlt_t,
      Wxp1, r2(bxp1), Wxp2, r2(bxp2), Wep, r2(bep),
      Wsl1[:F], Wsl1[F:], r2(bsl1), Wsl2, r2(bsl2), Wvl)
    hvec = jnp.transpose(hvt, (1, 0, 2))

    sl, vlo, ld = pl.pallas_call(
        _graph_body,
        out_shape=[
            jax.ShapeDtypeStruct((B, F), f32),
            jax.ShapeDtypeStruct((3, B, F), f32),
            jax.ShapeDtypeStruct((3, B, 1), f32),
        ],
    )(psx, psv, pcnt, scalar_l, vlt_t,
      Wsg1[:F], Wsg1[F:], r2(bsg1), Wsg2, r2(bsg2), Wvg,
      Wvlp, Wslp1[:F], Wslp1[F:], r2(bslp1), Wslp2, r2(bslp2), Wl)
    vl = jnp.transpose(vlo, (1, 0, 2))
    l_delta = jnp.transpose(ld, (1, 0, 2))

    return (hx, hvec, sl, vl, l_delta)


# speedup vs baseline: 27.5812x; 1.0105x over previous
import math

import jax
import jax.numpy as jnp
from jax.experimental import pallas as pl
from jax.experimental.pallas import tpu as pltpu

_SCALE = 1.0 / 0.6
_INV3 = 1.0 / math.sqrt(3.0)
_PREC = jax.lax.Precision.DEFAULT


def _ssilu(v):
    return jax.nn.silu(v) * _SCALE


def _dot(a, b):
    return jax.lax.dot_general(
        a, b, (((1,), (0,)), ((), ())),
        preferred_element_type=jnp.float32, precision=_PREC)


def _dot_tl(a, b):
    return jax.lax.dot_general(
        a, b, (((0,), (0,)), ((), ())),
        preferred_element_type=jnp.float32, precision=_PREC)


def _node_body(x_ref, vec_ref, ef_ref, eu_ref, btr_ref,
               slt_ref, vlt_ref,
               wxp1_ref, bxp1_ref, wxp2_ref, bxp2_ref, wep_ref, bep_ref,
               wsl1a_ref, wsl1b_ref, bsl1_ref, wsl2_ref, bsl2_ref, wvl_ref,
               wvli_ref,
               hx_ref, hvt_ref, psx_ref, psv_ref, pcnt_ref):
    f32 = jnp.float32
    blk, F = x_ref.shape
    B = slt_ref.shape[0]
    invh = 1.0 / math.sqrt(float(F))

    xb = x_ref[...]
    ids_r = btr_ref[0]
    onehot_t = (jax.lax.broadcasted_iota(jnp.int32, (B, 1), 0) == ids_r
                ).astype(f32)

    t = jax.nn.silu(_dot(xb, wxp1_ref[...]) + bxp1_ref[...])
    x_p = _dot(t, wxp2_ref[...]) + bxp2_ref[...]
    edge_p = _dot(ef_ref[...], wep_ref[...]) + bep_ref[...]
    prod = x_p * edge_p
    x1 = prod[:, :F]
    x2 = prod[:, F:2 * F]
    xn = prod[:, 2 * F:] + xb

    slb = _dot(slt_ref[...], wsl1b_ref[...])
    h = jax.nn.silu(_dot(xn, wsl1a_ref[...]) + _dot_tl(onehot_t, slb)
                    + bsl1_ref[...])
    hx = _ssilu(_dot(h, wsl2_ref[...]) + bsl2_ref[...]) + xn
    hx_ref[...] = hx
    psx_ref[0] = _dot(onehot_t, hx)
    pcnt_ref[0] = _dot(onehot_t, jnp.ones((blk, 8), f32))

    wvli = wvli_ref[...]
    wvl = wvl_ref[...]
    for k in range(3):
        vk = vec_ref[k]
        uk = eu_ref[:, k:k + 1]
        vecn_k = x1 * vk + x2 * uk
        vlw = _dot(vlt_ref[k], wvl)
        hvk = _dot(vecn_k, wvli) + _dot_tl(onehot_t, vlw)
        hvt_ref[k] = hvk
        psv_ref[0, :, k * F:(k + 1) * F] = _dot(onehot_t, hvk)


def _graph_body(psx_ref, psv_ref, pcnt_ref, slt_ref, vlt_ref,
                wsg1a_ref, wsg1b_ref, bsg1_ref, wsg2_ref, bsg2_ref, wvg_ref,
                wvlp_ref, wslp1a_ref, wslp1b_ref, bslp1_ref,
                wslp2_ref, bslp2_ref, wl_ref,
                sl_ref, vlo_ref, ld_ref):
    B, F = slt_ref.shape[0], slt_ref.shape[1]
    sum_x = jnp.sum(psx_ref[...], axis=0)
    sum_v = jnp.sum(psv_ref[...], axis=0)
    cnt = jnp.sum(pcnt_ref[...], axis=0)
    rinv = 1.0 / jnp.maximum(cnt[:, 0:1], 1.0)

    scalar_l = slt_ref[...]
    mean_x = sum_x * rinv
    h = _ssilu(_dot(mean_x, wsg1a_ref[...]) + _dot(scalar_l, wsg1b_ref[...])
               + bsg1_ref[...])
    sl = scalar_l + _ssilu(_dot(h, wsg2_ref[...]) + bsg2_ref[...])

    wvg = wvg_ref[...]
    wvlp = wvlp_ref[...]
    vh1 = []
    vls = []
    vn2 = jnp.zeros((B, F), jnp.float32)
    for k in range(3):
        vl_k = vlt_ref[k]
        mean_vk = sum_v[:, k * F:(k + 1) * F] * rinv
        vlk = vl_k + _dot(mean_vk + vl_k, wvg)
        vls.append(vlk)
        vh = _dot(vlk, wvlp)
        vh1.append(vh[:, :F])
        vn2 = vn2 + vh[:, F:] * vh[:, F:]
    vnorm = jnp.sqrt(vn2 + 1e-8)

    sh = _dot(_ssilu(_dot(sl, wslp1a_ref[...]) + _dot(vnorm, wslp1b_ref[...])
                     + bslp1_ref[...]), wslp2_ref[...]) + bslp2_ref[...]
    sh1 = sh[:, :F]
    sh2 = sh[:, F:2 * F]
    gate = jnp.tanh(sh[:, 2 * F:])
    sl_ref[...] = sh2 + sl * gate
    wl = wl_ref[...]
    for k in range(3):
        vlo = sh1 * vh1[k] + vls[k]
        vlo_ref[k] = vlo
        ld_ref[k] = _dot(vlo, wl)


def kernel(x, scalar_l, vec, vector_l, edge_feat, edge_udiff, batch,
           Wxp1, bxp1, Wxp2, bxp2, Wep, bep, Wsl1, bsl1, Wsl2, bsl2, Wvl,
           Wsg1, bsg1, Wsg2, bsg2, Wvg, Wvlp, Wslp1, bslp1, Wslp2, bslp2, Wl):
    N, F = x.shape
    B = scalar_l.shape[0]
    R = edge_feat.shape[1]
    blk = 2000
    nb = N // blk

    vec_t = jnp.transpose(vec, (1, 0, 2))
    vlt_t = jnp.transpose(vector_l, (1, 0, 2))
    bt_row = batch.reshape(nb, 1, blk)
    r2 = lambda b: b.reshape(1, -1)

    invh = 1.0 / math.sqrt(float(F))
    colscale = jnp.concatenate([
        jnp.full((2 * F,), _INV3 * invh, f32 := jnp.float32),
        jnp.full((F,), _INV3, f32)])
    Wep_s = Wep * colscale[None, :]
    bep_s = bep * colscale
    Wxp2_s = Wxp2 * _SCALE
    Wsl2_s = Wsl2 * _SCALE
    WvlI = Wvl + jnp.eye(F, dtype=f32)

    rep = lambda shape: pl.BlockSpec(shape, lambda i: (0,) * len(shape))

    out_shape = [
        jax.ShapeDtypeStruct((N, F), f32),
        jax.ShapeDtypeStruct((3, N, F), f32),
        jax.ShapeDtypeStruct((nb, B, F), f32),
        jax.ShapeDtypeStruct((nb, B, 3 * F), f32),
        jax.ShapeDtypeStruct((nb, B, 8), f32),
    ]
    in_specs = [
        pl.BlockSpec((blk, F), lambda i: (i, 0)),
        pl.BlockSpec((3, blk, F), lambda i: (0, i, 0)),
        pl.BlockSpec((blk, R), lambda i: (i, 0)),
        pl.BlockSpec((blk, 3), lambda i: (i, 0)),
        pl.BlockSpec((1, 1, blk), lambda i: (i, 0, 0)),
        rep((B, F)), rep((3, B, F)),
        rep((F, F)), rep((1, F)), rep((F, 3 * F)), rep((1, 3 * F)),
        rep((R, 3 * F)), rep((1, 3 * F)),
        rep((F, F)), rep((F, F)), rep((1, F)),
        rep((F, F)), rep((1, F)), rep((F, F)), rep((F, F)),
    ]
    out_specs = [
        pl.BlockSpec((blk, F), lambda i: (i, 0)),
        pl.BlockSpec((3, blk, F), lambda i: (0, i, 0)),
        pl.BlockSpec((1, B, F), lambda i: (i, 0, 0)),
        pl.BlockSpec((1, B, 3 * F), lambda i: (i, 0, 0)),
        pl.BlockSpec((1, B, 8), lambda i: (i, 0, 0)),
    ]
    hx, hvt, psx, psv, pcnt = pl.pallas_call(
        _node_body,
        grid=(nb,),
        in_specs=in_specs,
        out_specs=out_specs,
        out_shape=out_shape,
        compiler_params=pltpu.CompilerParams(
            dimension_semantics=("parallel",)),
    )(x, vec_t, edge_feat, edge_udiff, bt_row,
      scalar_l, vlt_t,
      Wxp1, r2(bxp1), Wxp2_s, r2(bxp2), Wep_s, r2(bep_s),
      Wsl1[:F], Wsl1[F:], r2(bsl1), Wsl2_s, r2(bsl2), Wvl, WvlI)
    hvec = jnp.transpose(hvt, (1, 0, 2))

    sl, vlo, ld = pl.pallas_call(
        _graph_body,
        out_shape=[
            jax.ShapeDtypeStruct((B, F), f32),
            jax.ShapeDtypeStruct((3, B, F), f32),
            jax.ShapeDtypeStruct((3, B, 1), f32),
        ],
    )(psx, psv, pcnt, scalar_l, vlt_t,
      Wsg1[:F], Wsg1[F:], r2(bsg1), Wsg2, r2(bsg2), Wvg,
      Wvlp, Wslp1[:F], Wslp1[F:], r2(bslp1), Wslp2, r2(bslp2), Wl)
    vl = jnp.transpose(vlo, (1, 0, 2))
    l_delta = jnp.transpose(ld, (1, 0, 2))

    return (hx, hvec, sl, vl, l_delta)
